# baseline (speedup 1.0000x reference)
import functools

import jax
import jax.numpy as jnp
from jax import lax
from jax.experimental import pallas as pl
from jax.experimental.pallas import tpu as pltpu
from jax.experimental.pallas import tpu_sc as plsc

_NC = 2
_NS = 16
_L = 16


def _make_aggregate(N, Np, E, D, B):
    EPC = E // _NC
    EPT = EPC // _NS
    NB = EPT // B
    assert NB * B == EPT and B % 8 == 0 and B <= 128 and NB >= 2
    NPAIR = NB // 2
    ODD = NB % 2 == 1
    ROWS_PT = Np // _NS
    NFULL = ROWS_PT // B
    assert NFULL * B == ROWS_PT
    CH = D // _L

    mesh = plsc.VectorSubcoreMesh(core_axis_name="c", subcore_axis_name="s")

    @functools.partial(
        pl.kernel,
        out_type=jax.ShapeDtypeStruct((_NC * Np, D), jnp.float32),
        mesh=mesh,
        scratch_types=[
            pltpu.VMEM((B,), jnp.int32),
            pltpu.VMEM((B,), jnp.int32),
            pltpu.VMEM((B,), jnp.int32),
            pltpu.VMEM((B,), jnp.int32),
            pltpu.VMEM((B, D), jnp.float32),
            pltpu.VMEM((B, D), jnp.float32),
            pltpu.VMEM((B, D), jnp.float32),
            pltpu.VMEM((B, D), jnp.float32),
            pltpu.VMEM_SHARED((Np, D), jnp.float32),
            pltpu.SemaphoreType.DMA,
            pltpu.SemaphoreType.DMA,
            pltpu.SemaphoreType.DMA,
            pltpu.SemaphoreType.DMA,
        ],
    )
    def agg(x_hbm, src_hbm, dst_hbm, e_hbm, out_hbm,
            src0, dst0, src1, dst1, xr0, er0, xr1, er1, acc,
            isem0, isem1, gsem0, gsem1):
        c = lax.axis_index("c")
        s = lax.axis_index("s")
        slots = ((src0, dst0, xr0, er0, isem0, gsem0),
                 (src1, dst1, xr1, er1, isem1, gsem1))
        ebase = c * EPC + s * EPT

        def eoff(i):
            return pl.multiple_of(ebase + i * B, 8)

        def issue_idx(slot, i):
            srcv, dstv, _, _, isem, _ = slots[slot]
            eb = eoff(i)
            pltpu.async_copy(src_hbm.at[pl.ds(eb, B)], srcv, isem)
            pltpu.async_copy(dst_hbm.at[pl.ds(eb, B)], dstv, isem)

        def wait_idx(slot):
            srcv, dstv, _, _, isem, _ = slots[slot]
            pltpu.make_async_copy(src_hbm.at[pl.ds(0, B)], srcv, isem).wait()
            pltpu.make_async_copy(dst_hbm.at[pl.ds(0, B)], dstv, isem).wait()

        def issue_gather(slot, i):
            srcv, _, xrv, erv, _, gsem = slots[slot]
            pltpu.async_copy(x_hbm.at[srcv], xrv, gsem)
            pltpu.async_copy(e_hbm.at[pl.ds(eoff(i), B)], erv, gsem)

        def wait_gather(slot):
            _, _, xrv, erv, _, gsem = slots[slot]
            pltpu.make_async_copy(x_hbm.at[pl.ds(0, B)], xrv, gsem).wait()
            pltpu.make_async_copy(e_hbm.at[pl.ds(0, B)], erv, gsem).wait()

        def compute_scatter(slot):
            _, dstv, xrv, erv, _, _ = slots[slot]

            def relu_row(r, _):
                for k in range(CH):
                    sl = pl.ds(k * _L, _L)
                    xrv[r, sl] = jnp.maximum(xrv[r, sl] + erv[r, sl], 0.0)
                return 0

            lax.fori_loop(0, B, relu_row, 0)
            pltpu.sync_copy(xrv, acc.at[dstv], add=True)

        zero = jnp.zeros((_L,), jnp.float32)

        def zfill(r, _):
            for k in range(CH):
                xr0[r, pl.ds(k * _L, _L)] = zero
            return 0

        lax.fori_loop(0, B, zfill, 0)
        row0 = s * ROWS_PT

        def zcopy(j, _):
            r = pl.multiple_of(row0 + j * B, 8)
            pltpu.sync_copy(xr0, acc.at[pl.ds(r, B)])
            return 0

        lax.fori_loop(0, NFULL, zcopy, 0)
        plsc.subcore_barrier()

        pltpu.sync_copy(src_hbm.at[pl.ds(eoff(0), B)], src0)
        pltpu.sync_copy(dst_hbm.at[pl.ds(eoff(0), B)], dst0)
        issue_gather(0, 0)
        issue_idx(1, jnp.minimum(1, NB - 1))

        def last(i):
            return jnp.minimum(i, NB - 1)

        def pair(j, _):
            a = 2 * j
            wait_idx(1)
            issue_gather(1, last(a + 1))
            wait_gather(0)
            compute_scatter(0)
            issue_idx(0, last(a + 2))
            wait_idx(0)
            issue_gather(0, last(a + 2))
            wait_gather(1)
            compute_scatter(1)
            issue_idx(1, last(a + 3))
            return 0

        lax.fori_loop(0, NPAIR, pair, 0)
        wait_idx(1)
        wait_gather(0)
        if ODD:
            compute_scatter(0)
        plsc.subcore_barrier()

        obase = c * Np + s * ROWS_PT

        def wout(j, _):
            r = pl.multiple_of(row0 + j * B, 8)
            o = pl.multiple_of(obase + j * B, 8)
            pltpu.sync_copy(acc.at[pl.ds(r, B)], xr0)
            pltpu.sync_copy(xr0, out_hbm.at[pl.ds(o, B)])
            return 0

        lax.fori_loop(0, NFULL, wout, 0)

    return agg


def _edge_mlp(edge_attr, Wcat, bcat, D):
    E, K = edge_attr.shape
    BE = 6400
    F = Wcat.shape[1]
    NO = F // D

    def body(a_ref, w_ref, b_ref, *o_refs):
        res = jnp.dot(a_ref[...], w_ref[...],
                      preferred_element_type=jnp.float32) + b_ref[...]
        for t, o_ref in enumerate(o_refs):
            o_ref[...] = res[:, t * D:(t + 1) * D]

    out = pl.pallas_call(
        body,
        grid=(E // BE,),
        in_specs=[
            pl.BlockSpec((BE, K), lambda i: (i, 0)),
            pl.BlockSpec((K, F), lambda i: (0, 0)),
            pl.BlockSpec((1, F), lambda i: (0, 0)),
        ],
        out_specs=[pl.BlockSpec((BE, D), lambda i: (i, 0))] * NO,
        out_shape=[jax.ShapeDtypeStruct((E, D), jnp.float32)] * NO,
    )(edge_attr, Wcat, bcat)
    return out


def _h_mlp(x, p0, p1, W1, b1, W2, b2):
    N, D = x.shape
    BN = 2000

    def body(x_ref, p0_ref, p1_ref, w1_ref, b1_ref, w2_ref, b2_ref, o_ref):
        hin = x_ref[...] + p0_ref[...] + p1_ref[...]
        t = jnp.maximum(
            jnp.dot(hin, w1_ref[...],
                    preferred_element_type=jnp.float32) + b1_ref[...], 0.0)
        o_ref[...] = jnp.maximum(
            jnp.dot(t, w2_ref[...],
                    preferred_element_type=jnp.float32) + b2_ref[...], 0.0)

    H = W1.shape[1]
    row = lambda i: (i, 0)
    full = lambda i: (0, 0)
    return pl.pallas_call(
        body,
        grid=(N // BN,),
        in_specs=[
            pl.BlockSpec((BN, D), row),
            pl.BlockSpec((BN, D), row),
            pl.BlockSpec((BN, D), row),
            pl.BlockSpec((D, H), full),
            pl.BlockSpec((1, H), full),
            pl.BlockSpec((H, H), full),
            pl.BlockSpec((1, H), full),
        ],
        out_specs=pl.BlockSpec((BN, H), row),
        out_shape=jax.ShapeDtypeStruct((N, H), jnp.float32),
    )(x, p0, p1, W1, b1, W2, b2)


def _head_mlp(h, pmu0, pmu1, pls0, pls1, mu_W1, mu_b1, mu_W2, mu_b2,
              ls_W1, ls_b1, ls_W2, ls_b2):
    N, D = h.shape
    LAT = mu_W1.shape[1]
    BN = 2000

    def body(h_ref, pmu0r, pmu1r, pls0r, pls1r, mw1, mb1, mw2, mb2,
             lw1, lb1, lw2, lb2, mu_ref, ls_ref):
        hmu = h_ref[...] + pmu0r[...] + pmu1r[...]
        t = jnp.maximum(
            jnp.dot(hmu, mw1[...],
                    preferred_element_type=jnp.float32) + mb1[...], 0.0)
        mu_ref[...] = jnp.dot(t, mw2[...],
                              preferred_element_type=jnp.float32) + mb2[...]
        hls = h_ref[...] + pls0r[...] + pls1r[...]
        u = jnp.maximum(
            jnp.dot(hls, lw1[...],
                    preferred_element_type=jnp.float32) + lb1[...], 0.0)
        ls = jnp.dot(u, lw2[...],
                     preferred_element_type=jnp.float32) + lb2[...]
        ls_ref[...] = jnp.clip(ls, -3.0, 3.0)

    row = lambda i: (i, 0)
    full = lambda i: (0, 0)
    return pl.pallas_call(
        body,
        grid=(N // BN,),
        in_specs=[pl.BlockSpec((BN, D), row)] * 5 + [
            pl.BlockSpec((D, LAT), full),
            pl.BlockSpec((1, LAT), full),
            pl.BlockSpec((LAT, LAT), full),
            pl.BlockSpec((1, LAT), full),
            pl.BlockSpec((D, LAT), full),
            pl.BlockSpec((1, LAT), full),
            pl.BlockSpec((LAT, LAT), full),
            pl.BlockSpec((1, LAT), full),
        ],
        out_specs=[pl.BlockSpec((BN, LAT), row)] * 2,
        out_shape=[jax.ShapeDtypeStruct((N, LAT), jnp.float32)] * 2,
    )(h, pmu0, pmu1, pls0, pls1, mu_W1, mu_b1, mu_W2, mu_b2,
      ls_W1, ls_b1, ls_W2, ls_b2)


def kernel(x, edge_index, edge_attr, lin1_W, lin1_b, nn1_W1, nn1_b1, nn1_W2,
           nn1_b2, linmu_W, linmu_b, mu_W1, mu_b1, mu_W2, mu_b2, linls_W,
           linls_b, ls_W1, ls_b1, ls_W2, ls_b2):
    N, D = x.shape
    E = edge_attr.shape[0]
    src = edge_index[0]
    dst = edge_index[1]

    (e1,) = _edge_mlp(edge_attr, lin1_W, lin1_b[None, :], D)

    Np = 10240
    agg = _make_aggregate(N, Np, E, D, B=80)
    parts1 = agg(x, src, dst, e1)

    Wcat = jnp.concatenate([linmu_W, linls_W], axis=1)
    bcat = jnp.concatenate([linmu_b, linls_b])[None, :]
    emu, els = _edge_mlp(edge_attr, Wcat, bcat, D)

    h = _h_mlp(x, parts1[:N], parts1[Np:Np + N],
               nn1_W1, nn1_b1[None, :], nn1_W2, nn1_b2[None, :])
    partsmu = agg(h, src, dst, emu)
    partsls = agg(h, src, dst, els)
    mu, logstd = _head_mlp(
        h, partsmu[:N], partsmu[Np:Np + N], partsls[:N], partsls[Np:Np + N],
        mu_W1, mu_b1[None, :], mu_W2, mu_b2[None, :],
        ls_W1, ls_b1[None, :], ls_W2, ls_b2[None, :])
    return (mu, logstd)


# speedup vs baseline: 1.0379x; 1.0379x over previous
import functools

import jax
import jax.numpy as jnp
import numpy as np
from jax import lax
from jax.experimental import pallas as pl
from jax.experimental.pallas import tpu as pltpu
from jax.experimental.pallas import tpu_sc as plsc

_NC = 2
_NS = 16
_L = 16


def _make_aggregate(N, Np, E, D, B):
    EPC = E // _NC
    EPT = EPC // _NS
    NB = EPT // B
    assert NB * B == EPT and B % 8 == 0 and B <= 128 and NB >= 2
    NPAIR = NB // 2
    ODD = NB % 2 == 1
    ROWS_PT = Np // _NS
    NFULL = ROWS_PT // B
    assert NFULL * B == ROWS_PT
    CH = D // _L

    mesh = plsc.VectorSubcoreMesh(core_axis_name="c", subcore_axis_name="s")

    @functools.partial(
        pl.kernel,
        out_type=jax.ShapeDtypeStruct((_NC * Np, D), jnp.float32),
        mesh=mesh,
        scratch_types=[
            pltpu.VMEM((B,), jnp.int32),
            pltpu.VMEM((B,), jnp.int32),
            pltpu.VMEM((B,), jnp.int32),
            pltpu.VMEM((B,), jnp.int32),
            pltpu.VMEM((B, D), jnp.float32),
            pltpu.VMEM((B, D), jnp.float32),
            pltpu.VMEM((B, D), jnp.float32),
            pltpu.VMEM((B, D), jnp.float32),
            pltpu.VMEM_SHARED((Np, D), jnp.float32),
            pltpu.SemaphoreType.DMA,
            pltpu.SemaphoreType.DMA,
            pltpu.SemaphoreType.DMA,
            pltpu.SemaphoreType.DMA,
        ],
    )
    def agg(x_hbm, src_hbm, dst_hbm, e_hbm, out_hbm,
            src0, dst0, src1, dst1, xr0, er0, xr1, er1, acc,
            isem0, isem1, gsem0, gsem1):
        c = lax.axis_index("c")
        s = lax.axis_index("s")
        slots = ((src0, dst0, xr0, er0, isem0, gsem0),
                 (src1, dst1, xr1, er1, isem1, gsem1))
        ebase = c * EPC + s * EPT

        def eoff(i):
            return pl.multiple_of(ebase + i * B, 8)

        def issue_idx(slot, i):
            srcv, dstv, _, _, isem, _ = slots[slot]
            eb = eoff(i)
            pltpu.async_copy(src_hbm.at[pl.ds(eb, B)], srcv, isem)
            pltpu.async_copy(dst_hbm.at[pl.ds(eb, B)], dstv, isem)

        def wait_idx(slot):
            srcv, dstv, _, _, isem, _ = slots[slot]
            pltpu.make_async_copy(src_hbm.at[pl.ds(0, B)], srcv, isem).wait()
            pltpu.make_async_copy(dst_hbm.at[pl.ds(0, B)], dstv, isem).wait()

        def issue_gather(slot, i):
            srcv, _, xrv, erv, _, gsem = slots[slot]
            pltpu.async_copy(x_hbm.at[srcv], xrv, gsem)
            pltpu.async_copy(e_hbm.at[pl.ds(eoff(i), B)], erv, gsem)

        def wait_gather(slot):
            _, _, xrv, erv, _, gsem = slots[slot]
            pltpu.make_async_copy(x_hbm.at[pl.ds(0, B)], xrv, gsem).wait()
            pltpu.make_async_copy(e_hbm.at[pl.ds(0, B)], erv, gsem).wait()

        def compute_scatter(slot):
            _, dstv, xrv, erv, _, _ = slots[slot]

            def relu_row(r, _):
                for k in range(CH):
                    sl = pl.ds(k * _L, _L)
                    xrv[r, sl] = jnp.maximum(xrv[r, sl] + erv[r, sl], 0.0)
                return 0

            lax.fori_loop(0, B, relu_row, 0)
            pltpu.sync_copy(xrv, acc.at[dstv], add=True)

        zero = jnp.zeros((_L,), jnp.float32)

        def zfill(r, _):
            for k in range(CH):
                xr0[r, pl.ds(k * _L, _L)] = zero
            return 0

        lax.fori_loop(0, B, zfill, 0)
        row0 = s * ROWS_PT

        def zcopy(j, _):
            r = pl.multiple_of(row0 + j * B, 8)
            pltpu.sync_copy(xr0, acc.at[pl.ds(r, B)])
            return 0

        lax.fori_loop(0, NFULL, zcopy, 0)
        plsc.subcore_barrier()

        pltpu.sync_copy(src_hbm.at[pl.ds(eoff(0), B)], src0)
        pltpu.sync_copy(dst_hbm.at[pl.ds(eoff(0), B)], dst0)
        issue_gather(0, 0)
        issue_idx(1, jnp.minimum(1, NB - 1))

        def last(i):
            return jnp.minimum(i, NB - 1)

        def pair(j, _):
            a = 2 * j
            wait_idx(1)
            issue_gather(1, last(a + 1))
            wait_gather(0)
            compute_scatter(0)
            issue_idx(0, last(a + 2))
            wait_idx(0)
            issue_gather(0, last(a + 2))
            wait_gather(1)
            compute_scatter(1)
            issue_idx(1, last(a + 3))
            return 0

        lax.fori_loop(0, NPAIR, pair, 0)
        wait_idx(1)
        wait_gather(0)
        if ODD:
            compute_scatter(0)
        plsc.subcore_barrier()

        obase = c * Np + s * ROWS_PT

        def wout(j, _):
            r = pl.multiple_of(row0 + j * B, 8)
            o = pl.multiple_of(obase + j * B, 8)
            pltpu.sync_copy(acc.at[pl.ds(r, B)], xr0)
            pltpu.sync_copy(xr0, out_hbm.at[pl.ds(o, B)])
            return 0

        lax.fori_loop(0, NFULL, wout, 0)

    return agg


def _pack_perm(D):
    perm = np.empty(D, np.int32)
    H = D // 2
    for j in range(H):
        perm[j] = 32 * (j // 16) + j % 16
        perm[H + j] = perm[j] + 16
    return perm


def _make_aggregate_pe(N, Np, E, D, B):
    EPC = E // _NC
    EPT = EPC // _NS
    NB = EPT // B
    assert NB * B == EPT and B % 8 == 0 and B <= 128 and NB >= 2
    NPAIR = NB // 2
    ODD = NB % 2 == 1
    ROWS_PT = Np // _NS
    NFULL = ROWS_PT // B
    assert NFULL * B == ROWS_PT
    G = D // 32
    Dw = D // 2

    mesh = plsc.VectorSubcoreMesh(core_axis_name="c", subcore_axis_name="s")

    @functools.partial(
        pl.kernel,
        out_type=jax.ShapeDtypeStruct((_NC * Np, D), jnp.float32),
        mesh=mesh,
        scratch_types=[
            pltpu.VMEM((B,), jnp.int32),
            pltpu.VMEM((B,), jnp.int32),
            pltpu.VMEM((B,), jnp.int32),
            pltpu.VMEM((B,), jnp.int32),
            pltpu.VMEM((B, D), jnp.float32),
            pltpu.VMEM((B, Dw), jnp.int32),
            pltpu.VMEM((B, D), jnp.float32),
            pltpu.VMEM((B, Dw), jnp.int32),
            pltpu.VMEM_SHARED((Np, D), jnp.float32),
            pltpu.SemaphoreType.DMA,
            pltpu.SemaphoreType.DMA,
            pltpu.SemaphoreType.DMA,
            pltpu.SemaphoreType.DMA,
        ],
    )
    def agg(x_hbm, src_hbm, dst_hbm, e_hbm, out_hbm,
            src0, dst0, src1, dst1, xr0, er0, xr1, er1, acc,
            isem0, isem1, gsem0, gsem1):
        c = lax.axis_index("c")
        s = lax.axis_index("s")
        slots = ((src0, dst0, xr0, er0, isem0, gsem0),
                 (src1, dst1, xr1, er1, isem1, gsem1))
        ebase = c * EPC + s * EPT

        def eoff(i):
            return pl.multiple_of(ebase + i * B, 8)

        def issue_idx(slot, i):
            srcv, dstv, _, _, isem, _ = slots[slot]
            eb = eoff(i)
            pltpu.async_copy(src_hbm.at[pl.ds(eb, B)], srcv, isem)
            pltpu.async_copy(dst_hbm.at[pl.ds(eb, B)], dstv, isem)

        def wait_idx(slot):
            srcv, dstv, _, _, isem, _ = slots[slot]
            pltpu.make_async_copy(src_hbm.at[pl.ds(0, B)], srcv, isem).wait()
            pltpu.make_async_copy(dst_hbm.at[pl.ds(0, B)], dstv, isem).wait()

        def issue_gather(slot, i):
            srcv, _, xrv, erv, _, gsem = slots[slot]
            pltpu.async_copy(x_hbm.at[srcv], xrv, gsem)
            pltpu.async_copy(e_hbm.at[pl.ds(eoff(i), B)], erv, gsem)

        def wait_gather(slot):
            _, _, xrv, erv, _, gsem = slots[slot]
            pltpu.make_async_copy(x_hbm.at[pl.ds(0, B)], xrv, gsem).wait()
            pltpu.make_async_copy(e_hbm.at[pl.ds(0, B)], erv, gsem).wait()

        himask = jnp.int32(-65536)

        def compute_scatter(slot):
            _, dstv, xrv, erv, _, _ = slots[slot]

            def relu_row(r, _):
                for g in range(G):
                    we = erv[r, pl.ds(16 * g, 16)]
                    elo = lax.bitcast_convert_type(jnp.left_shift(we, 16),
                                                   jnp.float32)
                    ehi = lax.bitcast_convert_type(we & himask, jnp.float32)
                    slo = pl.ds(32 * g, _L)
                    shi = pl.ds(32 * g + _L, _L)
                    xrv[r, slo] = jnp.maximum(xrv[r, slo] + elo, 0.0)
                    xrv[r, shi] = jnp.maximum(xrv[r, shi] + ehi, 0.0)
                return 0

            lax.fori_loop(0, B, relu_row, 0)
            pltpu.sync_copy(xrv, acc.at[dstv], add=True)

        zero = jnp.zeros((_L,), jnp.float32)

        def zfill(r, _):
            for k in range(D // _L):
                xr0[r, pl.ds(k * _L, _L)] = zero
            return 0

        lax.fori_loop(0, B, zfill, 0)
        row0 = s * ROWS_PT

        def zcopy(j, _):
            r = pl.multiple_of(row0 + j * B, 8)
            pltpu.sync_copy(xr0, acc.at[pl.ds(r, B)])
            return 0

        lax.fori_loop(0, NFULL, zcopy, 0)
        plsc.subcore_barrier()

        pltpu.sync_copy(src_hbm.at[pl.ds(eoff(0), B)], src0)
        pltpu.sync_copy(dst_hbm.at[pl.ds(eoff(0), B)], dst0)
        issue_gather(0, 0)
        issue_idx(1, jnp.minimum(1, NB - 1))

        def last(i):
            return jnp.minimum(i, NB - 1)

        def pair(j, _):
            a = 2 * j
            wait_idx(1)
            issue_gather(1, last(a + 1))
            wait_gather(0)
            compute_scatter(0)
            issue_idx(0, last(a + 2))
            wait_idx(0)
            issue_gather(0, last(a + 2))
            wait_gather(1)
            compute_scatter(1)
            issue_idx(1, last(a + 3))
            return 0

        lax.fori_loop(0, NPAIR, pair, 0)
        wait_idx(1)
        wait_gather(0)
        if ODD:
            compute_scatter(0)
        plsc.subcore_barrier()

        obase = c * Np + s * ROWS_PT

        def wout(j, _):
            r = pl.multiple_of(row0 + j * B, 8)
            o = pl.multiple_of(obase + j * B, 8)
            pltpu.sync_copy(acc.at[pl.ds(r, B)], xr0)
            pltpu.sync_copy(xr0, out_hbm.at[pl.ds(o, B)])
            return 0

        lax.fori_loop(0, NFULL, wout, 0)

    return agg


def _edge_mlp(edge_attr, Wcat, bcat, D, out_dtype=jnp.float32):
    E, K = edge_attr.shape
    BE = 6400
    F = Wcat.shape[1]
    NO = F // D

    def body(a_ref, w_ref, b_ref, *o_refs):
        res = jnp.dot(a_ref[...], w_ref[...],
                      preferred_element_type=jnp.float32) + b_ref[...]
        for t, o_ref in enumerate(o_refs):
            o_ref[...] = res[:, t * D:(t + 1) * D].astype(out_dtype)

    out = pl.pallas_call(
        body,
        grid=(E // BE,),
        in_specs=[
            pl.BlockSpec((BE, K), lambda i: (i, 0)),
            pl.BlockSpec((K, F), lambda i: (0, 0)),
            pl.BlockSpec((1, F), lambda i: (0, 0)),
        ],
        out_specs=[pl.BlockSpec((BE, D), lambda i: (i, 0))] * NO,
        out_shape=[jax.ShapeDtypeStruct((E, D), out_dtype)] * NO,
    )(edge_attr, Wcat, bcat)
    return out


def _edge_mlp_packed(edge_attr, Wcat, bcat, D):
    E, K = edge_attr.shape
    BE = 6400
    F = Wcat.shape[1]
    NO = F // D
    H = D // 2

    def body(a_ref, w_ref, b_ref, *o_refs):
        res = jnp.dot(a_ref[...], w_ref[...],
                      preferred_element_type=jnp.float32) + b_ref[...]
        u = lax.bitcast_convert_type(res, jnp.uint32)
        r = u + jnp.uint32(0x7FFF) + ((u >> 16) & jnp.uint32(1))
        for t, o_ref in enumerate(o_refs):
            lo = r[:, t * D:t * D + H] >> 16
            hi = r[:, t * D + H:(t + 1) * D] & jnp.uint32(0xFFFF0000)
            o_ref[...] = lax.bitcast_convert_type(lo | hi, jnp.int32)

    out = pl.pallas_call(
        body,
        grid=(E // BE,),
        in_specs=[
            pl.BlockSpec((BE, K), lambda i: (i, 0)),
            pl.BlockSpec((K, F), lambda i: (0, 0)),
            pl.BlockSpec((1, F), lambda i: (0, 0)),
        ],
        out_specs=[pl.BlockSpec((BE, H), lambda i: (i, 0))] * NO,
        out_shape=[jax.ShapeDtypeStruct((E, H), jnp.int32)] * NO,
    )(edge_attr, Wcat, bcat)
    return out


def _h_mlp(x, p0, p1, W1, b1, W2, b2):
    N, D = x.shape
    BN = 2000

    def body(x_ref, p0_ref, p1_ref, w1_ref, b1_ref, w2_ref, b2_ref, o_ref):
        hin = x_ref[...] + p0_ref[...] + p1_ref[...]
        t = jnp.maximum(
            jnp.dot(hin, w1_ref[...],
                    preferred_element_type=jnp.float32) + b1_ref[...], 0.0)
        o_ref[...] = jnp.maximum(
            jnp.dot(t, w2_ref[...],
                    preferred_element_type=jnp.float32) + b2_ref[...], 0.0)

    H = W1.shape[1]
    row = lambda i: (i, 0)
    full = lambda i: (0, 0)
    return pl.pallas_call(
        body,
        grid=(N // BN,),
        in_specs=[
            pl.BlockSpec((BN, D), row),
            pl.BlockSpec((BN, D), row),
            pl.BlockSpec((BN, D), row),
            pl.BlockSpec((D, H), full),
            pl.BlockSpec((1, H), full),
            pl.BlockSpec((H, H), full),
            pl.BlockSpec((1, H), full),
        ],
        out_specs=pl.BlockSpec((BN, H), row),
        out_shape=jax.ShapeDtypeStruct((N, H), jnp.float32),
    )(x, p0, p1, W1, b1, W2, b2)


def _head_mlp(h, pmu0, pmu1, pls0, pls1, mu_W1, mu_b1, mu_W2, mu_b2,
              ls_W1, ls_b1, ls_W2, ls_b2):
    N, D = h.shape
    LAT = mu_W1.shape[1]
    BN = 2000

    def body(h_ref, pmu0r, pmu1r, pls0r, pls1r, mw1, mb1, mw2, mb2,
             lw1, lb1, lw2, lb2, mu_ref, ls_ref):
        hmu = h_ref[...] + pmu0r[...] + pmu1r[...]
        t = jnp.maximum(
            jnp.dot(hmu, mw1[...],
                    preferred_element_type=jnp.float32) + mb1[...], 0.0)
        mu_ref[...] = jnp.dot(t, mw2[...],
                              preferred_element_type=jnp.float32) + mb2[...]
        hls = h_ref[...] + pls0r[...] + pls1r[...]
        u = jnp.maximum(
            jnp.dot(hls, lw1[...],
                    preferred_element_type=jnp.float32) + lb1[...], 0.0)
        ls = jnp.dot(u, lw2[...],
                     preferred_element_type=jnp.float32) + lb2[...]
        ls_ref[...] = jnp.clip(ls, -3.0, 3.0)

    row = lambda i: (i, 0)
    full = lambda i: (0, 0)
    return pl.pallas_call(
        body,
        grid=(N // BN,),
        in_specs=[pl.BlockSpec((BN, D), row)] * 5 + [
            pl.BlockSpec((D, LAT), full),
            pl.BlockSpec((1, LAT), full),
            pl.BlockSpec((LAT, LAT), full),
            pl.BlockSpec((1, LAT), full),
            pl.BlockSpec((D, LAT), full),
            pl.BlockSpec((1, LAT), full),
            pl.BlockSpec((LAT, LAT), full),
            pl.BlockSpec((1, LAT), full),
        ],
        out_specs=[pl.BlockSpec((BN, LAT), row)] * 2,
        out_shape=[jax.ShapeDtypeStruct((N, LAT), jnp.float32)] * 2,
    )(h, pmu0, pmu1, pls0, pls1, mu_W1, mu_b1, mu_W2, mu_b2,
      ls_W1, ls_b1, ls_W2, ls_b2)


def kernel(x, edge_index, edge_attr, lin1_W, lin1_b, nn1_W1, nn1_b1, nn1_W2,
           nn1_b2, linmu_W, linmu_b, mu_W1, mu_b1, mu_W2, mu_b2, linls_W,
           linls_b, ls_W1, ls_b1, ls_W2, ls_b2):
    N, D = x.shape
    E = edge_attr.shape[0]
    src = edge_index[0]
    dst = edge_index[1]

    (e1,) = _edge_mlp(edge_attr, lin1_W, lin1_b[None, :], D)

    Np = 10240
    agg = _make_aggregate(N, Np, E, D, B=80)
    parts1 = agg(x, src, dst, e1)

    perm = _pack_perm(D)
    Wcat = jnp.concatenate([linmu_W[:, perm], linls_W[:, perm]], axis=1)
    bcat = jnp.concatenate([linmu_b[perm], linls_b[perm]])[None, :]
    emu, els = _edge_mlp_packed(edge_attr, Wcat, bcat, D)

    h = _h_mlp(x, parts1[:N], parts1[Np:Np + N],
               nn1_W1, nn1_b1[None, :], nn1_W2, nn1_b2[None, :])
    aggb = _make_aggregate_pe(N, Np, E, D, B=80)
    partsmu = aggb(h, src, dst, emu)
    partsls = aggb(h, src, dst, els)
    mu, logstd = _head_mlp(
        h, partsmu[:N], partsmu[Np:Np + N], partsls[:N], partsls[Np:Np + N],
        mu_W1, mu_b1[None, :], mu_W2, mu_b2[None, :],
        ls_W1, ls_b1[None, :], ls_W2, ls_b2[None, :])
    return (mu, logstd)
